# baseline (speedup 1.0000x reference)
import jax
import jax.numpy as jnp
from jax import lax
from jax.experimental import pallas as pl
from jax.experimental.pallas import tpu as pltpu
from jax.experimental.pallas import tpu_sc as plsc

N_NODES = 10000
D = 128
N_GRAPHS = 128
FC = 128
NC = 2

NP = 10240
E0 = 320000
E_REAL = E0 + N_NODES
NWORK = 32
CH = 48
NCHUNK = 216
W_PER = CH * NCHUNK
E_PAD = NWORK * W_PER
ROWS_PER_TILE = NP // 16

_BLK = 1024
_GRID = NP // _BLK


def _sc_edge_body(xl_hbm, xr_hbm, idx_hbm, att_hbm, zrows_hbm,
                  zvec_hbm, num_hbm, den_hbm,
                  acc, den, ib0, ib1, ib2, ib3, xlb0, xlb1, xrb0, xrb1,
                  outb0, outb1, wb0, wb1, trow, att_v,
                  gsem0, gsem1, ssem0, ssem1, isem0, isem1):
    c = lax.axis_index("c")
    s = lax.axis_index("s")
    wid = s * 2 + c
    row0 = s * ROWS_PER_TILE

    ib = (ib0, ib1, ib2, ib3)
    xlb = (xlb0, xlb1)
    xrb = (xrb0, xrb1)
    outb = (outb0, outb1)
    wb = (wb0, wb1)
    gsem = (gsem0, gsem1)
    ssem = (ssem0, ssem1)
    isem = (isem0, isem1)

    pltpu.sync_copy(zrows_hbm, acc.at[pl.ds(row0, ROWS_PER_TILE), :])
    pltpu.sync_copy(zvec_hbm, den.at[pl.ds(row0, ROWS_PER_TILE)])
    pltpu.sync_copy(att_hbm, att_v)
    plsc.subcore_barrier()

    attv = [att_v[pl.ds(16 * k, 16)] for k in range(8)]
    rowbase = lax.iota(jnp.int32, 16) * 16

    def fetch_idx_start(ci, q):
        pltpu.make_async_copy(idx_hbm.at[wid, ci], ib[q], isem[q % 2]).start()

    def fetch_idx_wait(ci, q):
        pltpu.make_async_copy(idx_hbm.at[wid, ci], ib[q], isem[q % 2]).wait()

    def gather_start(q, b):
        pltpu.make_async_copy(xl_hbm.at[ib[q].at[0]], xlb[b], gsem[b]).start()
        pltpu.make_async_copy(xr_hbm.at[ib[q].at[1]], xrb[b], gsem[b]).start()

    def gather_wait(q, b):
        pltpu.make_async_copy(xl_hbm.at[ib[q].at[0]], xlb[b], gsem[b]).wait()
        pltpu.make_async_copy(xr_hbm.at[ib[q].at[1]], xrb[b], gsem[b]).wait()

    def scatter_start(q, b):
        pltpu.make_async_copy(outb[b], acc.at[ib[q].at[1]],
                              ssem[b]).start(add=True)
        pltpu.make_async_copy(wb[b], den.at[ib[q].at[1]],
                              ssem[b]).start(add=True)

    def scatter_wait(q, b):
        pltpu.make_async_copy(outb[b], acc.at[ib[q].at[1]], ssem[b]).wait()
        pltpu.make_async_copy(wb[b], den.at[ib[q].at[1]], ssem[b]).wait()

    def compute(b):
        xlb_b, xrb_b, outb_b, wb_b = xlb[b], xrb[b], outb[b], wb[b]

        def group(g, _):
            g16 = g * 16
            for e16 in range(16):
                e = g16 + e16
                t0 = None
                t1 = None
                for k in range(8):
                    a = xlb_b[e, pl.ds(16 * k, 16)]
                    bv = xrb_b[e, pl.ds(16 * k, 16)]
                    v = a + bv
                    u = jnp.maximum(v, 0.2 * v) * attv[k]
                    if k % 2 == 0:
                        t0 = u if t0 is None else t0 + u
                    else:
                        t1 = u if t1 is None else t1 + u
                trow[pl.ds(e16 * 16, 16)] = t0 + t1
            alphav = None
            for j in range(16):
                col = plsc.load_gather(trow, [rowbase + j])
                alphav = col if alphav is None else alphav + col
            wv = jnp.exp(alphav)
            wb_b[pl.ds(g16, 16)] = wv
            for e16 in range(16):
                e = g16 + e16
                w = wv[e16]
                for k in range(8):
                    outb_b[e, pl.ds(16 * k, 16)] = (
                        xlb_b[e, pl.ds(16 * k, 16)] * w)
            return ()

        lax.fori_loop(0, CH // 16, group, ())

    fetch_idx_start(0, 0)
    fetch_idx_start(1, 1)
    fetch_idx_wait(0, 0)
    gather_start(0, 0)

    def quad(t, _):
        for b4 in range(4):
            ci = 4 * t + b4
            b = b4 % 2
            q = b4
            qn = (b4 + 1) % 4
            qn2 = (b4 + 2) % 4

            @pl.when(ci >= 2)
            def _():
                scatter_wait(q, b)

            @pl.when(ci + 2 < NCHUNK)
            def _():
                fetch_idx_start(ci + 2, qn2)

            @pl.when(ci + 1 < NCHUNK)
            def _():
                fetch_idx_wait(ci + 1, qn)
                gather_start(qn, 1 - b)

            gather_wait(q, b)
            compute(b)
            scatter_start(q, b)
        return ()

    lax.fori_loop(0, NCHUNK // 4, quad, ())
    scatter_wait(2, 0)
    scatter_wait(3, 1)

    plsc.subcore_barrier()
    pltpu.sync_copy(acc.at[pl.ds(row0, ROWS_PER_TILE), :],
                    num_hbm.at[c, pl.ds(row0, ROWS_PER_TILE), :])
    pltpu.sync_copy(den.at[pl.ds(row0, ROWS_PER_TILE)],
                    den_hbm.at[c, pl.ds(row0, ROWS_PER_TILE)])


@jax.jit
def _sc_edge_layer(xl, xr, idx, att, zrows, zvec):
    mesh = plsc.VectorSubcoreMesh(core_axis_name="c", subcore_axis_name="s")
    return pl.kernel(
        _sc_edge_body,
        out_type=(
            jax.ShapeDtypeStruct((2, NP, D), jnp.float32),
            jax.ShapeDtypeStruct((2, NP), jnp.float32),
        ),
        mesh=mesh,
        compiler_params=pltpu.CompilerParams(needs_layout_passes=False),
        scratch_types=[
            pltpu.VMEM_SHARED((NP, D), jnp.float32),
            pltpu.VMEM_SHARED((NP,), jnp.float32),
            pltpu.VMEM((2, CH), jnp.int32),
            pltpu.VMEM((2, CH), jnp.int32),
            pltpu.VMEM((2, CH), jnp.int32),
            pltpu.VMEM((2, CH), jnp.int32),
            pltpu.VMEM((CH, D), jnp.float32),
            pltpu.VMEM((CH, D), jnp.float32),
            pltpu.VMEM((CH, D), jnp.float32),
            pltpu.VMEM((CH, D), jnp.float32),
            pltpu.VMEM((CH, D), jnp.float32),
            pltpu.VMEM((CH, D), jnp.float32),
            pltpu.VMEM((CH,), jnp.float32),
            pltpu.VMEM((CH,), jnp.float32),
            pltpu.VMEM((256,), jnp.float32),
            pltpu.VMEM((D,), jnp.float32),
            pltpu.SemaphoreType.DMA,
            pltpu.SemaphoreType.DMA,
            pltpu.SemaphoreType.DMA,
            pltpu.SemaphoreType.DMA,
            pltpu.SemaphoreType.DMA,
            pltpu.SemaphoreType.DMA,
        ],
    )(xl, xr, idx, att, zrows, zvec)


def _k1_body(x_ref, wl_ref, bl_ref, wr_ref, br_ref, xl_ref, xr_ref):
    xb = x_ref[...]
    xl_ref[...] = jnp.dot(xb, wl_ref[...],
                          preferred_element_type=jnp.float32) + bl_ref[...]
    xr_ref[...] = jnp.dot(xb, wr_ref[...],
                          preferred_element_type=jnp.float32) + br_ref[...]


@jax.jit
def _tc_transform(x, wl, bl, wr, br):
    return pl.pallas_call(
        _k1_body,
        grid=(_GRID,),
        in_specs=[
            pl.BlockSpec((_BLK, D), lambda i: (i, 0)),
            pl.BlockSpec((D, D), lambda i: (0, 0)),
            pl.BlockSpec((1, D), lambda i: (0, 0)),
            pl.BlockSpec((D, D), lambda i: (0, 0)),
            pl.BlockSpec((1, D), lambda i: (0, 0)),
        ],
        out_specs=[
            pl.BlockSpec((_BLK, D), lambda i: (i, 0)),
            pl.BlockSpec((_BLK, D), lambda i: (i, 0)),
        ],
        out_shape=[
            jax.ShapeDtypeStruct((NP, D), jnp.float32),
            jax.ShapeDtypeStruct((NP, D), jnp.float32),
        ],
    )(x, wl, bl, wr, br)


def _combine(num_ref, d0_ref, d1_ref, bias_ref):
    n = num_ref[0] + num_ref[1]
    d = d0_ref[...] + d1_ref[...]
    v = n / (d + 1e-16) + bias_ref[...]
    return jnp.where(v > 0.0, v, jnp.exp(v) - 1.0)


def _k2_body(num_ref, d0_ref, d1_ref, bias_ref, wl_ref, bl_ref, wr_ref,
             br_ref, xl_ref, xr_ref):
    h = _combine(num_ref, d0_ref, d1_ref, bias_ref)
    xl_ref[...] = jnp.dot(h, wl_ref[...],
                          preferred_element_type=jnp.float32) + bl_ref[...]
    xr_ref[...] = jnp.dot(h, wr_ref[...],
                          preferred_element_type=jnp.float32) + br_ref[...]


@jax.jit
def _tc_combine_transform(num, d0, d1, bias, wl, bl, wr, br):
    return pl.pallas_call(
        _k2_body,
        grid=(_GRID,),
        in_specs=[
            pl.BlockSpec((2, _BLK, D), lambda i: (0, i, 0)),
            pl.BlockSpec((_BLK, 1), lambda i: (i, 0)),
            pl.BlockSpec((_BLK, 1), lambda i: (i, 0)),
            pl.BlockSpec((1, D), lambda i: (0, 0)),
            pl.BlockSpec((D, D), lambda i: (0, 0)),
            pl.BlockSpec((1, D), lambda i: (0, 0)),
            pl.BlockSpec((D, D), lambda i: (0, 0)),
            pl.BlockSpec((1, D), lambda i: (0, 0)),
        ],
        out_specs=[
            pl.BlockSpec((_BLK, D), lambda i: (i, 0)),
            pl.BlockSpec((_BLK, D), lambda i: (i, 0)),
        ],
        out_shape=[
            jax.ShapeDtypeStruct((NP, D), jnp.float32),
            jax.ShapeDtypeStruct((NP, D), jnp.float32),
        ],
    )(num, d0, d1, bias, wl, bl, wr, br)


def _k3_body(num_ref, d0_ref, d1_ref, bias_ref, batch_ref, wf1_ref, bf1_ref,
             wf2_ref, bf2_ref, out_ref, pooled_acc, cnt_acc):
    i = pl.program_id(0)
    h = _combine(num_ref, d0_ref, d1_ref, bias_ref)
    bb = batch_ref[...]
    gids = lax.broadcasted_iota(jnp.int32, (1, N_GRAPHS), 1)
    onehot = (bb == gids).astype(jnp.float32)
    dn = (((0,), (0,)), ((), ()))
    psum = lax.dot_general(onehot, h, dn,
                           preferred_element_type=jnp.float32)
    csum = lax.dot_general(onehot, jnp.ones((_BLK, D), jnp.float32), dn,
                           preferred_element_type=jnp.float32)

    @pl.when(i == 0)
    def _():
        pooled_acc[...] = jnp.zeros_like(pooled_acc)
        cnt_acc[...] = jnp.zeros_like(cnt_acc)

    pooled_acc[...] += psum
    cnt_acc[...] += csum

    @pl.when(i == _GRID - 1)
    def _():
        pooled = pooled_acc[...] / jnp.maximum(cnt_acc[...], 1.0)
        z1 = jnp.dot(pooled, wf1_ref[...],
                     preferred_element_type=jnp.float32) + bf1_ref[...]
        z1 = jnp.maximum(z1, 0.0)
        z2 = jnp.dot(z1, wf2_ref[...],
                     preferred_element_type=jnp.float32) + bf2_ref[...]
        col = lax.broadcasted_iota(jnp.int32, (N_GRAPHS, FC), 1)
        zm = jnp.where(col < NC, z2, -1e30)
        m = jnp.max(zm, axis=1, keepdims=True)
        lse = jnp.log(jnp.sum(jnp.exp(zm - m), axis=1, keepdims=True))
        out_ref[...] = z2 - m - lse


@jax.jit
def _tc_pool_head(num, d0, d1, bias, batch2d, wf1, bf1, wf2, bf2):
    return pl.pallas_call(
        _k3_body,
        grid=(_GRID,),
        in_specs=[
            pl.BlockSpec((2, _BLK, D), lambda i: (0, i, 0)),
            pl.BlockSpec((_BLK, 1), lambda i: (i, 0)),
            pl.BlockSpec((_BLK, 1), lambda i: (i, 0)),
            pl.BlockSpec((1, D), lambda i: (0, 0)),
            pl.BlockSpec((_BLK, 1), lambda i: (i, 0)),
            pl.BlockSpec((D, FC), lambda i: (0, 0)),
            pl.BlockSpec((1, FC), lambda i: (0, 0)),
            pl.BlockSpec((FC, FC), lambda i: (0, 0)),
            pl.BlockSpec((1, FC), lambda i: (0, 0)),
        ],
        out_specs=pl.BlockSpec((N_GRAPHS, FC), lambda i: (0, 0)),
        out_shape=jax.ShapeDtypeStruct((N_GRAPHS, FC), jnp.float32),
        scratch_shapes=[
            pltpu.VMEM((N_GRAPHS, FC), jnp.float32),
            pltpu.VMEM((N_GRAPHS, FC), jnp.float32),
        ],
    )(num, d0, d1, bias, batch2d, wf1, bf1, wf2, bf2)


def kernel(x, edge_index, batch, Wl1, bl1, Wr1, br1, att1, bias1,
           Wl2, bl2, Wr2, br2, att2, bias2, Wf1, bf1, Wf2, bf2):
    f32 = jnp.float32
    npad = NP - N_NODES
    epad = E_PAD - E_REAL
    loop = jnp.arange(N_NODES, dtype=jnp.int32)
    src = jnp.concatenate([edge_index[0].astype(jnp.int32), loop,
                           jnp.arange(epad, dtype=jnp.int32) % N_NODES])
    dst = jnp.concatenate([edge_index[1].astype(jnp.int32), loop,
                           N_NODES + jnp.arange(epad, dtype=jnp.int32) % 240])
    xp = jnp.concatenate([x.astype(f32),
                          jnp.zeros((npad, D), f32)], axis=0)
    batch2d = jnp.concatenate(
        [batch.astype(jnp.int32),
         jnp.full((npad,), N_GRAPHS, jnp.int32)]).reshape(NP, 1)
    zrows = jnp.zeros((ROWS_PER_TILE, D), f32)
    zvec = jnp.zeros((ROWS_PER_TILE,), f32)
    wf2p = jnp.concatenate([Wf2.astype(f32),
                            jnp.zeros((FC, FC - NC), f32)], axis=1)
    bf2p = jnp.concatenate([bf2.astype(f32),
                            jnp.zeros((FC - NC,), f32)]).reshape(1, FC)

    idx = jnp.stack([src.reshape(NWORK, NCHUNK, CH),
                     dst.reshape(NWORK, NCHUNK, CH)], axis=2)
    xl1, xr1 = _tc_transform(xp, Wl1, bl1.reshape(1, D), Wr1,
                             br1.reshape(1, D))
    num1, den1 = _sc_edge_layer(xl1, xr1, idx, att1, zrows, zvec)
    xl2, xr2 = _tc_combine_transform(
        num1, den1[0].reshape(NP, 1), den1[1].reshape(NP, 1),
        bias1.reshape(1, D), Wl2, bl2.reshape(1, D), Wr2, br2.reshape(1, D))
    num2, den2 = _sc_edge_layer(xl2, xr2, idx, att2, zrows, zvec)
    out = _tc_pool_head(
        num2, den2[0].reshape(NP, 1), den2[1].reshape(NP, 1),
        bias2.reshape(1, D), batch2d, Wf1, bf1.reshape(1, FC), wf2p, bf2p)
    return out[:, :NC]


# speedup vs baseline: 1.1228x; 1.1228x over previous
import jax
import jax.numpy as jnp
from jax import lax
from jax.experimental import pallas as pl
from jax.experimental.pallas import tpu as pltpu
from jax.experimental.pallas import tpu_sc as plsc

N_NODES = 10000
D = 128
N_GRAPHS = 128
FC = 128
NC = 2

NP = 10240
E0 = 320000
E_REAL = E0 + N_NODES
NWORK = 32
CH = 48
NCHUNK = 216
W_PER = CH * NCHUNK
E_PAD = NWORK * W_PER
ROWS_PER_TILE = NP // 16

_BLK = 1024
_GRID = NP // _BLK


def _sc_edge_body(xl_hbm, xr_hbm, idx_hbm, att_hbm, zrows_hbm,
                  zvec_hbm, num_hbm, den_hbm,
                  acc, den, ib0, ib1, ib2, ib3, xlb0, xlb1, xrb0, xrb1,
                  outb0, outb1, wb0, wb1, trow, att_v,
                  gsem0, gsem1, ssem0, ssem1, isem0, isem1):
    c = lax.axis_index("c")
    s = lax.axis_index("s")
    wid = s * 2 + c
    row0 = s * ROWS_PER_TILE

    ib = (ib0, ib1, ib2, ib3)
    xlb = (xlb0, xlb1)
    xrb = (xrb0, xrb1)
    outb = (outb0, outb1)
    wb = (wb0, wb1)
    gsem = (gsem0, gsem1)
    ssem = (ssem0, ssem1)
    isem = (isem0, isem1)

    pltpu.sync_copy(zrows_hbm, acc.at[pl.ds(row0, ROWS_PER_TILE), :])
    pltpu.sync_copy(zvec_hbm, den.at[pl.ds(row0, ROWS_PER_TILE)])
    pltpu.sync_copy(att_hbm, att_v)
    plsc.subcore_barrier()

    attv = [att_v[pl.ds(16 * k, 16)] for k in range(8)]
    rowbase = lax.iota(jnp.int32, 16) * 16

    def fetch_idx_start(ci, q):
        pltpu.make_async_copy(idx_hbm.at[wid, ci], ib[q], isem[q % 2]).start()

    def fetch_idx_wait(ci, q):
        pltpu.make_async_copy(idx_hbm.at[wid, ci], ib[q], isem[q % 2]).wait()

    def gather_start(q, b):
        pltpu.make_async_copy(xl_hbm.at[ib[q].at[0]], xlb[b], gsem[b]).start()
        pltpu.make_async_copy(xr_hbm.at[ib[q].at[1]], xrb[b], gsem[b]).start()

    def gather_wait(q, b):
        pltpu.make_async_copy(xl_hbm.at[ib[q].at[0]], xlb[b], gsem[b]).wait()
        pltpu.make_async_copy(xr_hbm.at[ib[q].at[1]], xrb[b], gsem[b]).wait()

    def scatter_start(q, b):
        pltpu.make_async_copy(outb[b], acc.at[ib[q].at[1]],
                              ssem[b]).start(add=True)
        pltpu.make_async_copy(wb[b], den.at[ib[q].at[1]],
                              ssem[b]).start(add=True)

    def scatter_wait(q, b):
        pltpu.make_async_copy(outb[b], acc.at[ib[q].at[1]], ssem[b]).wait()
        pltpu.make_async_copy(wb[b], den.at[ib[q].at[1]], ssem[b]).wait()

    def compute(b):
        xlb_b, xrb_b, outb_b, wb_b = xlb[b], xrb[b], outb[b], wb[b]

        def group(g, _):
            g16 = g * 16
            for e16 in range(16):
                e = g16 + e16
                t = None
                for k in range(8):
                    a = xlb_b[e, pl.ds(16 * k, 16)]
                    bv = xrb_b[e, pl.ds(16 * k, 16)]
                    v = a + bv
                    u = jnp.maximum(v, 0.2 * v) * attv[k]
                    t = u if t is None else t + u
                trow[pl.ds(e16 * 16, 16)] = t
            alphav = None
            for j in range(16):
                col = plsc.load_gather(trow, [rowbase + j])
                alphav = col if alphav is None else alphav + col
            wv = jnp.exp(alphav)
            wb_b[pl.ds(g16, 16)] = wv
            for e16 in range(16):
                e = g16 + e16
                w = wv[e16]
                for k in range(8):
                    outb_b[e, pl.ds(16 * k, 16)] = (
                        xlb_b[e, pl.ds(16 * k, 16)] * w)
            return ()

        lax.fori_loop(0, CH // 16, group, ())

    fetch_idx_start(0, 0)
    fetch_idx_start(1, 1)
    fetch_idx_wait(0, 0)
    gather_start(0, 0)

    def quad(t, _):
        for b4 in range(4):
            ci = 4 * t + b4
            b = b4 % 2
            q = b4
            qn = (b4 + 1) % 4
            qn2 = (b4 + 2) % 4

            @pl.when(ci >= 2)
            def _():
                scatter_wait(q, b)

            @pl.when(ci + 2 < NCHUNK)
            def _():
                fetch_idx_start(ci + 2, qn2)

            @pl.when(ci + 1 < NCHUNK)
            def _():
                fetch_idx_wait(ci + 1, qn)
                gather_start(qn, 1 - b)

            gather_wait(q, b)
            compute(b)
            scatter_start(q, b)
        return ()

    lax.fori_loop(0, NCHUNK // 4, quad, ())
    scatter_wait(2, 0)
    scatter_wait(3, 1)

    plsc.subcore_barrier()
    pltpu.sync_copy(acc.at[pl.ds(row0, ROWS_PER_TILE), :],
                    num_hbm.at[c, pl.ds(row0, ROWS_PER_TILE), :])
    pltpu.sync_copy(den.at[pl.ds(row0, ROWS_PER_TILE)],
                    den_hbm.at[c, pl.ds(row0, ROWS_PER_TILE)])


@jax.jit
def _sc_edge_layer(xl, xr, idx, att, zrows, zvec):
    mesh = plsc.VectorSubcoreMesh(core_axis_name="c", subcore_axis_name="s")
    return pl.kernel(
        _sc_edge_body,
        out_type=(
            jax.ShapeDtypeStruct((2, NP, D), jnp.float32),
            jax.ShapeDtypeStruct((2, NP), jnp.float32),
        ),
        mesh=mesh,
        compiler_params=pltpu.CompilerParams(needs_layout_passes=False),
        scratch_types=[
            pltpu.VMEM_SHARED((NP, D), jnp.float32),
            pltpu.VMEM_SHARED((NP,), jnp.float32),
            pltpu.VMEM((2, CH), jnp.int32),
            pltpu.VMEM((2, CH), jnp.int32),
            pltpu.VMEM((2, CH), jnp.int32),
            pltpu.VMEM((2, CH), jnp.int32),
            pltpu.VMEM((CH, D), jnp.float32),
            pltpu.VMEM((CH, D), jnp.float32),
            pltpu.VMEM((CH, D), jnp.float32),
            pltpu.VMEM((CH, D), jnp.float32),
            pltpu.VMEM((CH, D), jnp.float32),
            pltpu.VMEM((CH, D), jnp.float32),
            pltpu.VMEM((CH,), jnp.float32),
            pltpu.VMEM((CH,), jnp.float32),
            pltpu.VMEM((256,), jnp.float32),
            pltpu.VMEM((D,), jnp.float32),
            pltpu.SemaphoreType.DMA,
            pltpu.SemaphoreType.DMA,
            pltpu.SemaphoreType.DMA,
            pltpu.SemaphoreType.DMA,
            pltpu.SemaphoreType.DMA,
            pltpu.SemaphoreType.DMA,
        ],
    )(xl, xr, idx, att, zrows, zvec)


def _k1_body(x_ref, wl_ref, bl_ref, wr_ref, br_ref, xl_ref, xr_ref):
    xb = x_ref[...]
    xl_ref[...] = jnp.dot(xb, wl_ref[...],
                          preferred_element_type=jnp.float32) + bl_ref[...]
    xr_ref[...] = jnp.dot(xb, wr_ref[...],
                          preferred_element_type=jnp.float32) + br_ref[...]


@jax.jit
def _tc_transform(x, wl, bl, wr, br):
    return pl.pallas_call(
        _k1_body,
        grid=(_GRID,),
        in_specs=[
            pl.BlockSpec((_BLK, D), lambda i: (i, 0)),
            pl.BlockSpec((D, D), lambda i: (0, 0)),
            pl.BlockSpec((1, D), lambda i: (0, 0)),
            pl.BlockSpec((D, D), lambda i: (0, 0)),
            pl.BlockSpec((1, D), lambda i: (0, 0)),
        ],
        out_specs=[
            pl.BlockSpec((_BLK, D), lambda i: (i, 0)),
            pl.BlockSpec((_BLK, D), lambda i: (i, 0)),
        ],
        out_shape=[
            jax.ShapeDtypeStruct((NP, D), jnp.float32),
            jax.ShapeDtypeStruct((NP, D), jnp.float32),
        ],
    )(x, wl, bl, wr, br)


def _combine(num_ref, d0_ref, d1_ref, bias_ref):
    n = num_ref[0] + num_ref[1]
    d = d0_ref[...] + d1_ref[...]
    v = n / (d + 1e-16) + bias_ref[...]
    return jnp.where(v > 0.0, v, jnp.exp(v) - 1.0)


def _k2_body(num_ref, d0_ref, d1_ref, bias_ref, wl_ref, bl_ref, wr_ref,
             br_ref, xl_ref, xr_ref):
    h = _combine(num_ref, d0_ref, d1_ref, bias_ref)
    xl_ref[...] = jnp.dot(h, wl_ref[...],
                          preferred_element_type=jnp.float32) + bl_ref[...]
    xr_ref[...] = jnp.dot(h, wr_ref[...],
                          preferred_element_type=jnp.float32) + br_ref[...]


@jax.jit
def _tc_combine_transform(num, d0, d1, bias, wl, bl, wr, br):
    return pl.pallas_call(
        _k2_body,
        grid=(_GRID,),
        in_specs=[
            pl.BlockSpec((2, _BLK, D), lambda i: (0, i, 0)),
            pl.BlockSpec((_BLK, 1), lambda i: (i, 0)),
            pl.BlockSpec((_BLK, 1), lambda i: (i, 0)),
            pl.BlockSpec((1, D), lambda i: (0, 0)),
            pl.BlockSpec((D, D), lambda i: (0, 0)),
            pl.BlockSpec((1, D), lambda i: (0, 0)),
            pl.BlockSpec((D, D), lambda i: (0, 0)),
            pl.BlockSpec((1, D), lambda i: (0, 0)),
        ],
        out_specs=[
            pl.BlockSpec((_BLK, D), lambda i: (i, 0)),
            pl.BlockSpec((_BLK, D), lambda i: (i, 0)),
        ],
        out_shape=[
            jax.ShapeDtypeStruct((NP, D), jnp.float32),
            jax.ShapeDtypeStruct((NP, D), jnp.float32),
        ],
    )(num, d0, d1, bias, wl, bl, wr, br)


def _k3_body(num_ref, d0_ref, d1_ref, bias_ref, batch_ref, wf1_ref, bf1_ref,
             wf2_ref, bf2_ref, out_ref, pooled_acc, cnt_acc):
    i = pl.program_id(0)
    h = _combine(num_ref, d0_ref, d1_ref, bias_ref)
    bb = batch_ref[...]
    gids = lax.broadcasted_iota(jnp.int32, (1, N_GRAPHS), 1)
    onehot = (bb == gids).astype(jnp.float32)
    dn = (((0,), (0,)), ((), ()))
    psum = lax.dot_general(onehot, h, dn,
                           preferred_element_type=jnp.float32)
    csum = lax.dot_general(onehot, jnp.ones((_BLK, D), jnp.float32), dn,
                           preferred_element_type=jnp.float32)

    @pl.when(i == 0)
    def _():
        pooled_acc[...] = jnp.zeros_like(pooled_acc)
        cnt_acc[...] = jnp.zeros_like(cnt_acc)

    pooled_acc[...] += psum
    cnt_acc[...] += csum

    @pl.when(i == _GRID - 1)
    def _():
        pooled = pooled_acc[...] / jnp.maximum(cnt_acc[...], 1.0)
        z1 = jnp.dot(pooled, wf1_ref[...],
                     preferred_element_type=jnp.float32) + bf1_ref[...]
        z1 = jnp.maximum(z1, 0.0)
        z2 = jnp.dot(z1, wf2_ref[...],
                     preferred_element_type=jnp.float32) + bf2_ref[...]
        col = lax.broadcasted_iota(jnp.int32, (N_GRAPHS, FC), 1)
        zm = jnp.where(col < NC, z2, -1e30)
        m = jnp.max(zm, axis=1, keepdims=True)
        lse = jnp.log(jnp.sum(jnp.exp(zm - m), axis=1, keepdims=True))
        out_ref[...] = z2 - m - lse


@jax.jit
def _tc_pool_head(num, d0, d1, bias, batch2d, wf1, bf1, wf2, bf2):
    return pl.pallas_call(
        _k3_body,
        grid=(_GRID,),
        in_specs=[
            pl.BlockSpec((2, _BLK, D), lambda i: (0, i, 0)),
            pl.BlockSpec((_BLK, 1), lambda i: (i, 0)),
            pl.BlockSpec((_BLK, 1), lambda i: (i, 0)),
            pl.BlockSpec((1, D), lambda i: (0, 0)),
            pl.BlockSpec((_BLK, 1), lambda i: (i, 0)),
            pl.BlockSpec((D, FC), lambda i: (0, 0)),
            pl.BlockSpec((1, FC), lambda i: (0, 0)),
            pl.BlockSpec((FC, FC), lambda i: (0, 0)),
            pl.BlockSpec((1, FC), lambda i: (0, 0)),
        ],
        out_specs=pl.BlockSpec((N_GRAPHS, FC), lambda i: (0, 0)),
        out_shape=jax.ShapeDtypeStruct((N_GRAPHS, FC), jnp.float32),
        scratch_shapes=[
            pltpu.VMEM((N_GRAPHS, FC), jnp.float32),
            pltpu.VMEM((N_GRAPHS, FC), jnp.float32),
        ],
    )(num, d0, d1, bias, batch2d, wf1, bf1, wf2, bf2)


def kernel(x, edge_index, batch, Wl1, bl1, Wr1, br1, att1, bias1,
           Wl2, bl2, Wr2, br2, att2, bias2, Wf1, bf1, Wf2, bf2):
    f32 = jnp.float32
    npad = NP - N_NODES
    epad = E_PAD - E_REAL
    loop = jnp.arange(N_NODES, dtype=jnp.int32)
    src = jnp.concatenate([edge_index[0].astype(jnp.int32), loop,
                           jnp.arange(epad, dtype=jnp.int32) % N_NODES])
    dst = jnp.concatenate([edge_index[1].astype(jnp.int32), loop,
                           N_NODES + jnp.arange(epad, dtype=jnp.int32) % 240])
    xp = jnp.concatenate([x.astype(f32),
                          jnp.zeros((npad, D), f32)], axis=0)
    batch2d = jnp.concatenate(
        [batch.astype(jnp.int32),
         jnp.full((npad,), N_GRAPHS, jnp.int32)]).reshape(NP, 1)
    zrows = jnp.zeros((ROWS_PER_TILE, D), f32)
    zvec = jnp.zeros((ROWS_PER_TILE,), f32)
    wf2p = jnp.concatenate([Wf2.astype(f32),
                            jnp.zeros((FC, FC - NC), f32)], axis=1)
    bf2p = jnp.concatenate([bf2.astype(f32),
                            jnp.zeros((FC - NC,), f32)]).reshape(1, FC)

    idx = jnp.stack([src.reshape(NWORK, NCHUNK, CH),
                     dst.reshape(NWORK, NCHUNK, CH)], axis=2)
    xl1, xr1 = _tc_transform(xp, Wl1, bl1.reshape(1, D), Wr1,
                             br1.reshape(1, D))
    num1, den1 = _sc_edge_layer(xl1, xr1, idx, att1, zrows, zvec)
    xl2, xr2 = _tc_combine_transform(
        num1, den1[0].reshape(NP, 1), den1[1].reshape(NP, 1),
        bias1.reshape(1, D), Wl2, bl2.reshape(1, D), Wr2, br2.reshape(1, D))
    num2, den2 = _sc_edge_layer(xl2, xr2, idx, att2, zrows, zvec)
    out = _tc_pool_head(
        num2, den2[0].reshape(NP, 1), den2[1].reshape(NP, 1),
        bias2.reshape(1, D), batch2d, Wf1, bf1.reshape(1, FC), wf2p, bf2p)
    return out[:, :NC]
